# Initial kernel scaffold; baseline (speedup 1.0000x reference)
import jax
import jax.numpy as jnp
from jax.experimental import pallas as pl


def kernel(x, edge_index_rel0, edge_index_rel1, w_rel0, w_rel1, loop_weight, h_bias):
    raise NotImplementedError("write your pallas kernel here")


# speedup vs baseline: 4.8509x; 4.8509x over previous
import functools

import jax
import jax.numpy as jnp
from jax import lax
from jax.experimental import pallas as pl
from jax.experimental.pallas import tpu as pltpu
from jax.experimental.pallas import tpu_sc as plsc

N = 10000
D = 256
E = 80000

L = 16
NSUB = 16
NCORE = 2
W = 128
HALF = N // NCORE
ACC_ROWS = HALF + 8
ACC_V = 2 * ACC_ROWS
SSEG_LEN = 5120
CH = 128
NCHUNK = E // CH
ITER_E = 40
RB = 8
NRB = HALF // RB
NZC = ACC_V // L
ITER_R = 40


def _dense_body(x_ref, w2_ref, lw_ref, b_ref, g_ref, base_ref):
    x = x_ref[...]
    scores = jnp.dot(x, w2_ref[...], preferred_element_type=jnp.float32)
    m = jnp.max(scores, axis=0, keepdims=True)
    g_ref[...] = jnp.exp(scores - m)
    base_ref[...] = (
        jnp.dot(x, lw_ref[...], preferred_element_type=jnp.float32) + b_ref[...]
    )


_dense = pl.pallas_call(
    _dense_body,
    out_shape=[
        jax.ShapeDtypeStruct((N, 8), jnp.float32),
        jax.ShapeDtypeStruct((N, D), jnp.float32),
    ],
)


def _sc_body(x_hbm, g0_hbm, g1_hbm, src0_hbm, dst0_hbm, src1_hbm, dst1_hbm,
             base_hbm, out_hbm,
             acc, sseg, srcbuf, dstbuf, idxa, idxb, gsbuf, xbufa, xbufb,
             zrow, zvec, rowacc, rowprev, sbuf):
    cid = lax.axis_index("c")
    sid = lax.axis_index("s")
    base_node = cid * HALF

    zv = jnp.zeros((L,), jnp.float32)
    for r in range(L):
        for k in range(W // L):
            zrow[r, pl.ds(k * L, L)] = zv

    def zero_vec_body(k, _):
        zvec[pl.ds(k * L, L)] = zv
        return 0

    lax.fori_loop(0, (SSEG_LEN // NSUB) // L, zero_vec_body, 0)

    for rel in range(2):
        g_hbm = (g0_hbm, g1_hbm)[rel]
        src_hbm = (src0_hbm, src1_hbm)[rel]
        dst_hbm = (dst0_hbm, dst1_hbm)[rel]
        prev_hbm = base_hbm if rel == 0 else out_hbm

        def zero_body(i, _):
            ch = sid + NSUB * i

            @pl.when(ch < NZC)
            def _():
                pltpu.sync_copy(zrow, acc.at[pl.ds(ch * L, L)])

            return 0

        lax.fori_loop(0, ITER_R, zero_body, 0)
        pltpu.sync_copy(zvec, sseg.at[pl.ds(sid * (SSEG_LEN // NSUB),
                                            SSEG_LEN // NSUB)])
        plsc.subcore_barrier()

        def edge_body(i, _):
            ch = sid + NSUB * i

            @pl.when(ch < NCHUNK)
            def _():
                ebase = ch * CH
                pltpu.sync_copy(src_hbm.at[pl.ds(ebase, CH)], srcbuf)
                pltpu.sync_copy(dst_hbm.at[pl.ds(ebase, CH)], dstbuf)
                pltpu.sync_copy(g_hbm.at[srcbuf], gsbuf)
                for k in range(CH // L):
                    sl = pl.ds(k * L, L)
                    sv2 = srcbuf[sl] * 2
                    idxa[sl] = sv2
                    idxb[sl] = sv2 + 1
                pltpu.sync_copy(x_hbm.at[idxa], xbufa)
                pltpu.sync_copy(x_hbm.at[idxb], xbufb)
                for k in range(CH // L):
                    sl = pl.ds(k * L, L)
                    dl = dstbuf[sl] - base_node
                    owned = (dl >= 0) & (dl < HALF)
                    dl = jnp.where(owned, dl, HALF)
                    idxa[sl] = dl
                pltpu.sync_copy(gsbuf, sseg.at[idxa], add=True)

                def scale_body(i2, _):
                    gv = gsbuf[pl.ds(i2 * L, L)]
                    for r in range(L):
                        w = jnp.full((L,), gv[r], jnp.float32)
                        row = i2 * L + r
                        for k in range(W // L):
                            sl = pl.ds(k * L, L)
                            xbufa[row, sl] = xbufa[row, sl] * w
                            xbufb[row, sl] = xbufb[row, sl] * w
                    return 0

                lax.fori_loop(0, CH // L, scale_body, 0)
                for k in range(CH // L):
                    sl = pl.ds(k * L, L)
                    dv2 = idxa[sl] * 2
                    idxa[sl] = dv2
                    idxb[sl] = dv2 + 1
                pltpu.sync_copy(xbufa, acc.at[idxa], add=True)
                pltpu.sync_copy(xbufb, acc.at[idxb], add=True)

            return 0

        lax.fori_loop(0, ITER_E, edge_body, 0)
        plsc.subcore_barrier()

        def rb_body(i, _):
            ch = sid + NSUB * i

            @pl.when(ch < NRB)
            def _():
                r0 = ch * RB
                v0 = 2 * r0
                g0 = cid * 2 * HALF + v0
                pltpu.sync_copy(acc.at[pl.ds(v0, L)], rowacc)
                pltpu.sync_copy(prev_hbm.at[pl.ds(g0, L)], rowprev)
                pltpu.sync_copy(sseg.at[pl.ds(r0, L)], sbuf)
                sv = sbuf[...]
                rv = 1.0 / jnp.maximum(sv, 1e-9)
                for r in range(L):
                    w = jnp.full((L,), rv[r // 2], jnp.float32)
                    for k in range(W // L):
                        sl = pl.ds(k * L, L)
                        rowprev[r, sl] = rowprev[r, sl] + w * rowacc[r, sl]
                pltpu.sync_copy(rowprev, out_hbm.at[pl.ds(g0, L)])

            return 0

        lax.fori_loop(0, ITER_R, rb_body, 0)
        if rel == 0:
            plsc.subcore_barrier()


_sc_agg = functools.partial(
    pl.kernel,
    out_type=jax.ShapeDtypeStruct((2 * N, W), jnp.float32),
    mesh=plsc.VectorSubcoreMesh(core_axis_name="c", subcore_axis_name="s"),
    scratch_types=[
        pltpu.VMEM_SHARED((ACC_V, W), jnp.float32),
        pltpu.VMEM_SHARED((SSEG_LEN,), jnp.float32),
        pltpu.VMEM((CH,), jnp.int32),
        pltpu.VMEM((CH,), jnp.int32),
        pltpu.VMEM((CH,), jnp.int32),
        pltpu.VMEM((CH,), jnp.int32),
        pltpu.VMEM((CH,), jnp.float32),
        pltpu.VMEM((CH, W), jnp.float32),
        pltpu.VMEM((CH, W), jnp.float32),
        pltpu.VMEM((L, W), jnp.float32),
        pltpu.VMEM((SSEG_LEN // NSUB,), jnp.float32),
        pltpu.VMEM((L, W), jnp.float32),
        pltpu.VMEM((L, W), jnp.float32),
        pltpu.VMEM((L,), jnp.float32),
    ],
)(_sc_body)


@jax.jit
def kernel(x, edge_index_rel0, edge_index_rel1, w_rel0, w_rel1, loop_weight,
           h_bias):
    w2 = jnp.zeros((D, 8), jnp.float32)
    w2 = w2.at[:, 0].set(w_rel0[:D]).at[:, 1].set(w_rel1[:D])
    g8, base = _dense(x, w2, loop_weight, h_bias.reshape(1, D))
    out2 = _sc_agg(
        x.reshape(2 * N, W),
        g8[:, 0],
        g8[:, 1],
        edge_index_rel0[0],
        edge_index_rel0[1],
        edge_index_rel1[0],
        edge_index_rel1[1],
        base.reshape(2 * N, W),
    )
    return out2.reshape(N, D)


# speedup vs baseline: 5.0594x; 1.0430x over previous
import functools

import jax
import jax.numpy as jnp
from jax import lax
from jax.experimental import pallas as pl
from jax.experimental.pallas import tpu as pltpu
from jax.experimental.pallas import tpu_sc as plsc

N = 10000
D = 256
E = 80000

L = 16
NSUB = 16
W = 128
CH = 128
E2 = 81920
NT = E2 // CH // NSUB
ACC_ROWS = 10240
ZR = 16
NZC = ACC_ROWS // ZR
NRB = N // ZR
ITER_R = 40


def _dense_body(x_ref, w2_ref, lw_ref, b_ref, g_ref, base_ref):
    x = x_ref[...]
    scores = jnp.dot(x, w2_ref[...], preferred_element_type=jnp.float32)
    m = jnp.max(scores, axis=0, keepdims=True)
    g_ref[...] = jnp.exp(scores - m)
    base_ref[...] = (
        jnp.dot(x, lw_ref[...], preferred_element_type=jnp.float32) + b_ref[...]
    )


_dense = pl.pallas_call(
    _dense_body,
    out_shape=[
        jax.ShapeDtypeStruct((N, 8), jnp.float32),
        jax.ShapeDtypeStruct((N, D), jnp.float32),
    ],
)


def _sc_body(xlo_hbm, xhi_hbm, g0_hbm, g1_hbm, src0_hbm, dst0_hbm,
             src1_hbm, dst1_hbm, blo_hbm, bhi_hbm, outlo_hbm, outhi_hbm,
             acc, sseg, srcbuf, dstbuf, gsbuf, xbuf,
             zrow, zvec, rowacc, rowprev, sbuf):
    cid = lax.axis_index("c")
    sid = lax.axis_index("s")

    zv = jnp.zeros((L,), jnp.float32)
    for r in range(ZR):
        for k in range(W // L):
            zrow[r, pl.ds(k * L, L)] = zv

    def zero_vec_body(k, _):
        zvec[pl.ds(k * L, L)] = zv
        return 0

    lax.fori_loop(0, (ACC_ROWS // NSUB) // L, zero_vec_body, 0)

    def run_core(xh_hbm, base_hbm, out_hbm):
        for rel in range(2):
            g_hbm = (g0_hbm, g1_hbm)[rel]
            src_hbm = (src0_hbm, src1_hbm)[rel]
            dst_hbm = (dst0_hbm, dst1_hbm)[rel]
            prev_hbm = base_hbm if rel == 0 else out_hbm

            def zero_body(i, _):
                ch = sid + NSUB * i
                pltpu.sync_copy(zrow, acc.at[pl.ds(ch * ZR, ZR)])
                return 0

            lax.fori_loop(0, NZC // NSUB, zero_body, 0)
            pltpu.sync_copy(zvec, sseg.at[pl.ds(sid * (ACC_ROWS // NSUB),
                                                ACC_ROWS // NSUB)])
            plsc.subcore_barrier()

            def edge_body(j, _):
                ebase = (sid * NT + j) * CH
                pltpu.sync_copy(src_hbm.at[pl.ds(ebase, CH)], srcbuf)
                pltpu.sync_copy(dst_hbm.at[pl.ds(ebase, CH)], dstbuf)
                pltpu.sync_copy(g_hbm.at[srcbuf], gsbuf)
                pltpu.sync_copy(xh_hbm.at[srcbuf], xbuf)
                pltpu.sync_copy(gsbuf, sseg.at[dstbuf], add=True)

                def scale_body(i2, _):
                    gv = gsbuf[pl.ds(i2 * L, L)]
                    for r in range(L):
                        w = jnp.full((L,), gv[r], jnp.float32)
                        row = i2 * L + r
                        for k in range(W // L):
                            sl = pl.ds(k * L, L)
                            xbuf[row, sl] = xbuf[row, sl] * w
                    return 0

                lax.fori_loop(0, CH // L, scale_body, 0)
                pltpu.sync_copy(xbuf, acc.at[dstbuf], add=True)
                return 0

            lax.fori_loop(0, NT, edge_body, 0)
            plsc.subcore_barrier()

            def rb_body(i, _):
                ch = sid + NSUB * i

                @pl.when(ch < NRB)
                def _():
                    r0 = ch * ZR
                    pltpu.sync_copy(acc.at[pl.ds(r0, ZR)], rowacc)
                    pltpu.sync_copy(prev_hbm.at[pl.ds(r0, ZR)], rowprev)
                    pltpu.sync_copy(sseg.at[pl.ds(r0, L)], sbuf)
                    rv = 1.0 / jnp.maximum(sbuf[...], 1e-9)
                    for r in range(ZR):
                        w = jnp.full((L,), rv[r], jnp.float32)
                        for k in range(W // L):
                            sl = pl.ds(k * L, L)
                            rowprev[r, sl] = rowprev[r, sl] + w * rowacc[r, sl]
                    pltpu.sync_copy(rowprev, out_hbm.at[pl.ds(r0, ZR)])

                return 0

            lax.fori_loop(0, ITER_R, rb_body, 0)

    @pl.when(cid == 0)
    def _():
        run_core(xlo_hbm, blo_hbm, outlo_hbm)

    @pl.when(cid == 1)
    def _():
        run_core(xhi_hbm, bhi_hbm, outhi_hbm)


_sc_agg = functools.partial(
    pl.kernel,
    out_type=[
        jax.ShapeDtypeStruct((N, W), jnp.float32),
        jax.ShapeDtypeStruct((N, W), jnp.float32),
    ],
    mesh=plsc.VectorSubcoreMesh(core_axis_name="c", subcore_axis_name="s"),
    scratch_types=[
        pltpu.VMEM_SHARED((ACC_ROWS, W), jnp.float32),
        pltpu.VMEM_SHARED((ACC_ROWS,), jnp.float32),
        pltpu.VMEM((CH,), jnp.int32),
        pltpu.VMEM((CH,), jnp.int32),
        pltpu.VMEM((CH,), jnp.float32),
        pltpu.VMEM((CH, W), jnp.float32),
        pltpu.VMEM((ZR, W), jnp.float32),
        pltpu.VMEM((ACC_ROWS // NSUB,), jnp.float32),
        pltpu.VMEM((ZR, W), jnp.float32),
        pltpu.VMEM((ZR, W), jnp.float32),
        pltpu.VMEM((L,), jnp.float32),
    ],
)(_sc_body)


@jax.jit
def kernel(x, edge_index_rel0, edge_index_rel1, w_rel0, w_rel1, loop_weight,
           h_bias):
    w2 = jnp.zeros((D, 8), jnp.float32)
    w2 = w2.at[:, 0].set(w_rel0[:D]).at[:, 1].set(w_rel1[:D])
    g8, base = _dense(x, w2, loop_weight, h_bias.reshape(1, D))
    pad_src = jnp.zeros((E2 - E,), jnp.int32)
    pad_dst = jnp.full((E2 - E,), N, jnp.int32)
    outlo, outhi = _sc_agg(
        x[:, :W],
        x[:, W:],
        g8[:, 0],
        g8[:, 1],
        jnp.concatenate([edge_index_rel0[0], pad_src]),
        jnp.concatenate([edge_index_rel0[1], pad_dst]),
        jnp.concatenate([edge_index_rel1[0], pad_src]),
        jnp.concatenate([edge_index_rel1[1], pad_dst]),
        base[:, :W],
        base[:, W:],
    )
    return jnp.concatenate([outlo, outhi], axis=1)


# speedup vs baseline: 7.0400x; 1.3915x over previous
import functools

import jax
import jax.numpy as jnp
from jax import lax
from jax.experimental import pallas as pl
from jax.experimental.pallas import tpu as pltpu
from jax.experimental.pallas import tpu_sc as plsc

N = 10000
D = 256
E = 80000

L = 16
NSUB = 16
W = 128
CH = 128
E2 = 81920
NT = E2 // CH // NSUB
ACC_ROWS = 10240
ZR = 16
RPT = ACC_ROWS // NSUB
NRC = RPT // ZR


def _dense_body(x_ref, w2_ref, lw_ref, b_ref, g_ref, base_ref):
    x = x_ref[...]
    scores = jnp.dot(x, w2_ref[...], preferred_element_type=jnp.float32)
    m = jnp.max(scores, axis=0, keepdims=True)
    g_ref[...] = jnp.exp(scores - m)
    base_ref[...] = (
        jnp.dot(x, lw_ref[...], preferred_element_type=jnp.float32) + b_ref[...]
    )


_dense = pl.pallas_call(
    _dense_body,
    out_shape=[
        jax.ShapeDtypeStruct((N, 8), jnp.float32),
        jax.ShapeDtypeStruct((N, D), jnp.float32),
    ],
)


def _sc_body(xlo_hbm, xhi_hbm, g0_hbm, g1_hbm, src0_hbm, dst0_hbm,
             src1_hbm, dst1_hbm, blo_hbm, bhi_hbm, outlo_hbm, outhi_hbm,
             acc, sseg,
             srcb0, srcb1, dstb0, dstb1, gsb0, gsb1, xb0, xb1,
             zrow, zvec, sall, rowacc, rowprev,
             gsem0, gsem1):
    cid = lax.axis_index("c")
    sid = lax.axis_index("s")

    srcb = (srcb0, srcb1)
    dstb = (dstb0, dstb1)
    gsb = (gsb0, gsb1)
    xb = (xb0, xb1)
    gsem = (gsem0, gsem1)

    zv = jnp.zeros((L,), jnp.float32)
    for r in range(ZR):
        for k in range(W // L):
            zrow[r, pl.ds(k * L, L)] = zv

    def zero_vec_body(k, _):
        zvec[pl.ds(k * L, L)] = zv
        return 0

    lax.fori_loop(0, RPT // L, zero_vec_body, 0)

    def run_core(xh_hbm, base_hbm, out_hbm):
        row_base = sid * RPT

        for rel in range(2):
            g_hbm = (g0_hbm, g1_hbm)[rel]
            src_hbm = (src0_hbm, src1_hbm)[rel]
            dst_hbm = (dst0_hbm, dst1_hbm)[rel]
            prev_hbm = base_hbm if rel == 0 else out_hbm

            def zero_body(i, _):
                pltpu.sync_copy(zrow, acc.at[pl.ds(row_base + i * ZR, ZR)])
                return 0

            lax.fori_loop(0, NRC, zero_body, 0)
            pltpu.sync_copy(zvec, sseg.at[pl.ds(row_base, RPT)])
            plsc.subcore_barrier()

            def fetch_ids(j, p):
                eb = (sid * NT + j) * CH
                pltpu.sync_copy(src_hbm.at[pl.ds(eb, CH)], srcb[p])
                pltpu.sync_copy(dst_hbm.at[pl.ds(eb, CH)], dstb[p])

            def issue_gathers(p):
                pltpu.async_copy(g_hbm.at[srcb[p]], gsb[p], gsem[p])
                pltpu.async_copy(xh_hbm.at[srcb[p]], xb[p], gsem[p])

            def wait_gathers(p):
                pltpu.make_async_copy(g_hbm.at[srcb[p]], gsb[p],
                                      gsem[p]).wait()
                pltpu.make_async_copy(xh_hbm.at[srcb[p]], xb[p],
                                      gsem[p]).wait()

            def process(p):
                def scale_body(i2, _):
                    gv = gsb[p][pl.ds(i2 * L, L)]
                    for r in range(L):
                        w = jnp.full((L,), gv[r], jnp.float32)
                        row = i2 * L + r
                        for k in range(W // L):
                            sl = pl.ds(k * L, L)
                            xb[p][row, sl] = xb[p][row, sl] * w
                    return 0

                lax.fori_loop(0, CH // L, scale_body, 0)
                pltpu.sync_copy(gsb[p], sseg.at[dstb[p]], add=True)
                pltpu.sync_copy(xb[p], acc.at[dstb[p]], add=True)

            fetch_ids(0, 0)
            issue_gathers(0)

            def edge_pair(jj, _):
                for b in (0, 1):
                    j = 2 * jj + b
                    p, q = b, 1 - b
                    if b == 0:
                        fetch_ids(j + 1, q)
                        issue_gathers(q)
                    else:
                        @pl.when(jj < NT // 2 - 1)
                        def _():
                            fetch_ids(j + 1, q)
                            issue_gathers(q)

                    wait_gathers(p)
                    process(p)
                return 0

            lax.fori_loop(0, NT // 2, edge_pair, 0)
            plsc.subcore_barrier()

            pltpu.sync_copy(sseg.at[pl.ds(row_base, RPT)], sall)

            def rb_body(c, _):
                r0 = row_base + c * ZR

                @pl.when(r0 < N)
                def _():
                    pltpu.sync_copy(acc.at[pl.ds(r0, ZR)], rowacc)
                    pltpu.sync_copy(prev_hbm.at[pl.ds(r0, ZR)], rowprev)
                    rvec = 1.0 / jnp.maximum(sall[pl.ds(c * ZR, L)], 1e-9)
                    for r in range(ZR):
                        w = jnp.full((L,), rvec[r], jnp.float32)
                        for k in range(W // L):
                            sl = pl.ds(k * L, L)
                            rowprev[r, sl] = (rowprev[r, sl]
                                              + w * rowacc[r, sl])
                    pltpu.sync_copy(rowprev, out_hbm.at[pl.ds(r0, ZR)])

                return 0

            lax.fori_loop(0, NRC, rb_body, 0)

    @pl.when(cid == 0)
    def _():
        run_core(xlo_hbm, blo_hbm, outlo_hbm)

    @pl.when(cid == 1)
    def _():
        run_core(xhi_hbm, bhi_hbm, outhi_hbm)


_sc_agg = functools.partial(
    pl.kernel,
    out_type=[
        jax.ShapeDtypeStruct((N, W), jnp.float32),
        jax.ShapeDtypeStruct((N, W), jnp.float32),
    ],
    mesh=plsc.VectorSubcoreMesh(core_axis_name="c", subcore_axis_name="s"),
    scratch_types=[
        pltpu.VMEM_SHARED((ACC_ROWS, W), jnp.float32),
        pltpu.VMEM_SHARED((ACC_ROWS,), jnp.float32),
        pltpu.VMEM((CH,), jnp.int32),
        pltpu.VMEM((CH,), jnp.int32),
        pltpu.VMEM((CH,), jnp.int32),
        pltpu.VMEM((CH,), jnp.int32),
        pltpu.VMEM((CH,), jnp.float32),
        pltpu.VMEM((CH,), jnp.float32),
        pltpu.VMEM((CH, W), jnp.float32),
        pltpu.VMEM((CH, W), jnp.float32),
        pltpu.VMEM((ZR, W), jnp.float32),
        pltpu.VMEM((RPT,), jnp.float32),
        pltpu.VMEM((RPT,), jnp.float32),
        pltpu.VMEM((ZR, W), jnp.float32),
        pltpu.VMEM((ZR, W), jnp.float32),
        pltpu.SemaphoreType.DMA,
        pltpu.SemaphoreType.DMA,
    ],
)(_sc_body)


@jax.jit
def kernel(x, edge_index_rel0, edge_index_rel1, w_rel0, w_rel1, loop_weight,
           h_bias):
    w2 = jnp.zeros((D, 8), jnp.float32)
    w2 = w2.at[:, 0].set(w_rel0[:D]).at[:, 1].set(w_rel1[:D])
    g8, base = _dense(x, w2, loop_weight, h_bias.reshape(1, D))
    pad_src = jnp.zeros((E2 - E,), jnp.int32)
    pad_dst = jnp.full((E2 - E,), N, jnp.int32)
    outlo, outhi = _sc_agg(
        x[:, :W],
        x[:, W:],
        g8[:, 0],
        g8[:, 1],
        jnp.concatenate([edge_index_rel0[0], pad_src]),
        jnp.concatenate([edge_index_rel0[1], pad_dst]),
        jnp.concatenate([edge_index_rel1[0], pad_src]),
        jnp.concatenate([edge_index_rel1[1], pad_dst]),
        base[:, :W],
        base[:, W:],
    )
    return jnp.concatenate([outlo, outhi], axis=1)
